# baseline (speedup 1.0000x reference)
import functools
import math

import jax
import jax.numpy as jnp
from jax import lax
from jax.experimental import pallas as pl
from jax.experimental.pallas import tpu as pltpu
from jax.experimental.pallas import tpu_sc as plsc

N = 10000
NFEAT = 128
NHID = 128
NCLASS = 64
DEPTH = 3
RATIO = 0.5
TEMP = 0.5
WEIGHT_CPC = 0.5
SUB = 2048

E = 320000
NC = 2
NS = 16
NW = NC * NS
EW = E // NW
K = 80
NCHUNK = EW // K


def _npad(n):
    m = 16 * K
    p = ((n + m) // m) * m
    return p


def _msg_body(npad, d, xwp_hbm, src_hbm, dst_hbm, ew_hbm, out_hbm,
              shared, zbuf, sidx, didx, ewb, dstx, rows, sem):
    c = lax.axis_index("c")
    s = lax.axis_index("s")
    wid = s * NC + c
    rpt = npad // 16
    nzc = rpt // K

    def zz(r, _):
        for j in range(d // 16):
            zbuf[r, pl.ds(j * 16, 16)] = jnp.zeros((16,), jnp.float32)
        return 0
    lax.fori_loop(0, K, zz, 0)

    for c8 in range(nzc):
        pltpu.sync_copy(zbuf, shared.at[pl.ds(s * rpt + c8 * K, K)])
    plsc.subcore_barrier()

    trash = jnp.int32(npad - 1)
    base = wid * EW

    def chunk(ci, _):
        off = base + ci * K
        pltpu.sync_copy(src_hbm.at[pl.ds(off, K)], sidx)
        pltpu.sync_copy(dst_hbm.at[pl.ds(off, K)], didx)
        pltpu.sync_copy(ew_hbm.at[pl.ds(off, K)], ewb)
        for g in range(K // 16):
            dd = didx[pl.ds(g * 16, 16)]
            w = ewb[pl.ds(g * 16, 16)]
            dstx[pl.ds(g * 16, 16)] = jnp.where(w > 0.0, dd, trash)
        pltpu.async_copy(xwp_hbm.at[sidx], rows, sem).wait()
        pltpu.sync_copy(rows, shared.at[dstx], add=True)
        return 0
    lax.fori_loop(0, NCHUNK, chunk, 0)
    plsc.subcore_barrier()

    for c8 in range(nzc):
        st = s * rpt + c8 * K
        pltpu.sync_copy(shared.at[pl.ds(st, K)], rows)
        pltpu.sync_copy(rows, out_hbm.at[c, pl.ds(st, K)])


@functools.partial(jax.jit, static_argnames=("n", "d"))
def _sc_msg(xwp, src, dst, ew, n, d):
    npad = _npad(n)
    mesh = plsc.VectorSubcoreMesh(core_axis_name="c", subcore_axis_name="s")
    body = functools.partial(_msg_body, npad, d)
    f = pl.kernel(
        body,
        out_type=jax.ShapeDtypeStruct((NC, npad, d), jnp.float32),
        mesh=mesh,
        scratch_types=[
            pltpu.VMEM_SHARED((npad, d), jnp.float32),
            pltpu.VMEM((K, d), jnp.float32),
            pltpu.VMEM((K,), jnp.int32),
            pltpu.VMEM((K,), jnp.int32),
            pltpu.VMEM((K,), jnp.float32),
            pltpu.VMEM((K,), jnp.int32),
            pltpu.VMEM((K, d), jnp.float32),
            pltpu.SemaphoreType.DMA,
        ],
    )
    return f(xwp, src, dst, ew)


def _mms_body(x_ref, w_ref, dis_ref, o_ref):
    acc = jnp.dot(x_ref[...], w_ref[...], preferred_element_type=jnp.float32)
    o_ref[...] = acc * dis_ref[...][:, None]


def _matmul_scale(x, w, dis, block=512):
    n, k = x.shape
    ko, m = w.shape
    grid = (pl.cdiv(n, block),)
    return pl.pallas_call(
        _mms_body,
        grid=grid,
        in_specs=[
            pl.BlockSpec((block, k), lambda i: (i, 0)),
            pl.BlockSpec((ko, m), lambda i: (0, 0)),
            pl.BlockSpec((block,), lambda i: (i,)),
        ],
        out_specs=pl.BlockSpec((block, m), lambda i: (i, 0)),
        out_shape=jax.ShapeDtypeStruct((n, m), jnp.float32),
    )(x, w, dis)


def _comb_body(p0_ref, p1_ref, xwp_ref, dis_ref, b_ref, o_ref, *, relu):
    acc = (p0_ref[...] + p1_ref[...] + xwp_ref[...]) * dis_ref[...][:, None]
    acc = acc + b_ref[...][None, :]
    if relu:
        acc = jnp.maximum(acc, 0.0)
    o_ref[...] = acc


def _combine(p0, p1, xwp, dis, b, relu, block=512):
    n, m = xwp.shape
    grid = (pl.cdiv(n, block),)
    return pl.pallas_call(
        functools.partial(_comb_body, relu=relu),
        grid=grid,
        in_specs=[
            pl.BlockSpec((block, m), lambda i: (i, 0)),
            pl.BlockSpec((block, m), lambda i: (i, 0)),
            pl.BlockSpec((block, m), lambda i: (i, 0)),
            pl.BlockSpec((block,), lambda i: (i,)),
            pl.BlockSpec((m,), lambda i: (0,)),
        ],
        out_specs=pl.BlockSpec((block, m), lambda i: (i, 0)),
        out_shape=jax.ShapeDtypeStruct((n, m), jnp.float32),
    )(p0, p1, xwp, dis, b)


def _gcn(x, src, dst, ew, W, b, n, relu):
    d = W.shape[1]
    deg = jnp.zeros((n,), x.dtype).at[dst].add(ew) + 1.0
    dis = lax.rsqrt(deg)
    xwp = _matmul_scale(x, W, dis)
    part = _sc_msg(xwp, src, dst, ew, n=n, d=d)
    p0 = part[0, :n]
    p1 = part[1, :n]
    return _combine(p0, p1, xwp, dis, b, relu)


def _pool(x, src, dst, ew, p, k, n):
    score = (x @ p) / (jnp.linalg.norm(p) + 1e-12)
    vals, perm = jax.lax.top_k(score, k)
    x2 = x[perm] * jnp.tanh(vals)[:, None]
    keep = jnp.zeros((n,), dtype=bool).at[perm].set(True)
    newidx = jnp.zeros((n,), src.dtype).at[perm].set(jnp.arange(k, dtype=src.dtype))
    valid = keep[src] & keep[dst]
    src2 = jnp.where(valid, newidx[src], 0)
    dst2 = jnp.where(valid, newidx[dst], 0)
    ew2 = ew * valid.astype(x.dtype)
    return x2, src2, dst2, ew2, perm


def _supcon(feat, labels, temp):
    f = feat / (jnp.linalg.norm(feat, axis=1, keepdims=True) + 1e-12)
    sim = f @ f.T / temp
    m = feat.shape[0]
    eye = jnp.eye(m, dtype=bool)
    logits = sim - lax.stop_gradient(jnp.max(sim, axis=1, keepdims=True))
    expl = jnp.exp(logits) * (~eye)
    logprob = logits - jnp.log(jnp.sum(expl, axis=1, keepdims=True) + 1e-12)
    pos = (labels[:, None] == labels[None, :]) & (~eye)
    cnt = jnp.sum(pos, axis=1)
    mlpp = jnp.sum(jnp.where(pos, logprob, 0.0), axis=1) / jnp.maximum(cnt, 1)
    return -jnp.mean(jnp.where(cnt > 0, mlpp, 0.0))


def kernel(feature, edge_index, labels, idx_train, Wd0, bd0, Wd1, bd1, Wd2, bd2, Wd3, bd3, p0, p1, p2, Wu0, bu0, Wu1, bu1, Wu2, bu2):
    Wd = [Wd0, Wd1, Wd2, Wd3]; bd = [bd0, bd1, bd2, bd3]
    ps = [p0, p1, p2]
    Wu2p = jnp.pad(Wu2, ((0, 0), (0, NHID - NCLASS)))
    bu2p = jnp.pad(bu2, (0, NHID - NCLASS))
    Wu = [Wu0, Wu1, Wu2p]; bu = [bu0, bu1, bu2p]

    n = feature.shape[0]
    src, dst = edge_index[0], edge_index[1]
    ew = jnp.ones((src.shape[0],), feature.dtype)
    x = _gcn(feature, src, dst, ew, Wd[0], bd[0], n, relu=True)
    xs = [x]; srcs = [src]; dsts = [dst]; ews = [ew]; ns = [n]; perms = []
    for i in range(1, DEPTH + 1):
        k = int(math.ceil(RATIO * n))
        x, src, dst, ew, perm = _pool(x, src, dst, ew, ps[i - 1], k, n)
        n = k
        x = _gcn(x, src, dst, ew, Wd[i], bd[i], n, relu=True)
        if i < DEPTH:
            xs.append(x); srcs.append(src); dsts.append(dst); ews.append(ew); ns.append(n)
        perms.append(perm)
    for i in range(DEPTH):
        j = DEPTH - 1 - i
        res = xs[j]
        up = jnp.zeros_like(res).at[perms[j]].set(x)
        x = res + up
        x = _gcn(x, srcs[j], dsts[j], ews[j], Wu[i], bu[i], ns[j],
                 relu=(i < DEPTH - 1))

    x = x[:, :NCLASS]
    logits = x[idx_train]
    lt = labels[idx_train]
    logp = jax.nn.log_softmax(logits, axis=1)
    ce = -jnp.mean(jnp.take_along_axis(logp, lt[:, None], axis=1))
    scl = _supcon(logits, lt, TEMP)
    bcl = 0.0
    for i in range(DEPTH - 1):
        X = xs[0]; Y = xs[i + 1]
        Xd = lax.stop_gradient(X); Yd = lax.stop_gradient(Y)
        d2 = jnp.sum(Xd * Xd, 1)[:, None] - 2.0 * (Xd @ Yd.T) + jnp.sum(Yd * Yd, 1)[None, :]
        idx = jnp.argmin(d2, axis=1)
        cl = perms[i][idx]
        bcl = bcl + _supcon(X[:SUB], cl[:SUB], TEMP)
    return ce + (scl + bcl) * WEIGHT_CPC
